# Initial kernel scaffold; baseline (speedup 1.0000x reference)
import jax
import jax.numpy as jnp
from jax.experimental import pallas as pl


def kernel(idx, embedding):
    raise NotImplementedError("write your pallas kernel here")


# speedup vs baseline: 1.5776x; 1.5776x over previous
import functools

import jax
import jax.numpy as jnp
from jax import lax
from jax.experimental import pallas as pl
from jax.experimental.pallas import tpu as pltpu
from jax.experimental.pallas import tpu_sc as plsc

NC = 2
NS = 16
NW = NC * NS


@functools.lru_cache(maxsize=None)
def _make_gather(batch: int, vocab: int, dim: int, k: int):
    bpw = batch // NW
    nchunk = bpw // k
    mesh = plsc.VectorSubcoreMesh(
        core_axis_name="c", subcore_axis_name="s",
        num_cores=NC, num_subcores=NS,
    )

    @functools.partial(
        pl.kernel,
        out_type=jax.ShapeDtypeStruct((batch, dim), jnp.float32),
        mesh=mesh,
        scratch_types=[
            pltpu.VMEM((nchunk, k), jnp.int32),
            pltpu.VMEM((k, dim), jnp.float32),
            pltpu.SemaphoreType.DMA,
        ],
    )
    def gather_kernel(idx_hbm, table_hbm, out_hbm, idx_v, buf, gsem):
        wid = lax.axis_index("s") * NC + lax.axis_index("c")
        base = wid * bpw
        pltpu.sync_copy(idx_hbm.at[wid], idx_v)

        @pl.loop(0, nchunk)
        def _chunk(j):
            pltpu.async_copy(table_hbm.at[idx_v.at[j]], buf, gsem).wait()
            pltpu.sync_copy(buf, out_hbm.at[pl.ds(base + j * k, k)])

    return gather_kernel


def kernel(idx, embedding):
    if idx.ndim == 2:
        idx = jnp.squeeze(idx, axis=-1)
    batch = idx.shape[0]
    vocab, dim = embedding.shape
    k = 4
    idx3 = idx.astype(jnp.int32).reshape(NW, (batch // NW) // k, k)
    return _make_gather(batch, vocab, dim, k)(idx3, embedding)


# speedup vs baseline: 1.8017x; 1.1421x over previous
import functools

import jax
import jax.numpy as jnp
from jax import lax
from jax.experimental import pallas as pl
from jax.experimental.pallas import tpu as pltpu
from jax.experimental.pallas import tpu_sc as plsc

NC = 2
NS = 16
NW = NC * NS


@functools.lru_cache(maxsize=None)
def _make_gather(batch: int, vocab: int, dim: int, k: int):
    bpw = batch // NW
    nchunk = bpw // k
    mesh = plsc.VectorSubcoreMesh(
        core_axis_name="c", subcore_axis_name="s",
        num_cores=NC, num_subcores=NS,
    )

    @functools.partial(
        pl.kernel,
        out_type=jax.ShapeDtypeStruct((batch, dim), jnp.float32),
        mesh=mesh,
        scratch_types=[
            pltpu.VMEM((nchunk, k), jnp.int32),
            pltpu.VMEM((k, dim), jnp.float32),
            pltpu.VMEM((k, dim), jnp.float32),
            pltpu.SemaphoreType.DMA,
            pltpu.SemaphoreType.DMA,
            pltpu.SemaphoreType.DMA,
            pltpu.SemaphoreType.DMA,
        ],
    )
    def gather_kernel(idx_hbm, table_hbm, out_hbm, idx_v,
                      buf_a, buf_b, gs_a, gs_b, ss_a, ss_b):
        wid = lax.axis_index("s") * NC + lax.axis_index("c")
        base = wid * bpw
        pltpu.sync_copy(idx_hbm.at[wid], idx_v)

        def start_g(j, buf, sem):
            pltpu.async_copy(table_hbm.at[idx_v.at[j]], buf, sem)

        def wait_g(buf, sem):
            pltpu.make_async_copy(table_hbm.at[idx_v.at[0]], buf, sem).wait()

        def start_s(j, buf, sem):
            pltpu.async_copy(buf, out_hbm.at[pl.ds(base + j * k, k)], sem)

        def wait_s(buf, sem):
            pltpu.make_async_copy(buf, out_hbm.at[pl.ds(base, k)], sem).wait()

        start_g(0, buf_a, gs_a)
        start_g(1, buf_b, gs_b)

        @pl.loop(0, nchunk // 2)
        def _pair(i):
            j = i * 2
            wait_g(buf_a, gs_a)
            start_s(j, buf_a, ss_a)
            wait_g(buf_b, gs_b)
            start_s(j + 1, buf_b, ss_b)

            @pl.when(j + 2 < nchunk)
            def _refill():
                wait_s(buf_a, ss_a)
                start_g(j + 2, buf_a, gs_a)
                wait_s(buf_b, ss_b)
                start_g(j + 3, buf_b, gs_b)

        wait_s(buf_a, ss_a)
        wait_s(buf_b, ss_b)

    return gather_kernel


def kernel(idx, embedding):
    if idx.ndim == 2:
        idx = jnp.squeeze(idx, axis=-1)
    batch = idx.shape[0]
    vocab, dim = embedding.shape
    k = 4
    idx3 = idx.astype(jnp.int32).reshape(NW, (batch // NW) // k, k)
    return _make_gather(batch, vocab, dim, k)(idx3, embedding)


# speedup vs baseline: 1.8467x; 1.0250x over previous
import functools

import jax
import jax.numpy as jnp
from jax import lax
from jax.experimental import pallas as pl
from jax.experimental.pallas import tpu as pltpu
from jax.experimental.pallas import tpu_sc as plsc

NC = 2
NS = 16
NW = NC * NS


@functools.lru_cache(maxsize=None)
def _make_gather(batch: int, vocab: int, dim: int, k: int, ring: int):
    bpw = batch // NW
    nchunk = bpw // k
    assert nchunk % ring == 0
    mesh = plsc.VectorSubcoreMesh(
        core_axis_name="c", subcore_axis_name="s",
        num_cores=NC, num_subcores=NS,
    )

    @functools.partial(
        pl.kernel,
        out_type=jax.ShapeDtypeStruct((batch, dim), jnp.float32),
        mesh=mesh,
        scratch_types=[
            pltpu.VMEM((nchunk, k), jnp.int32),
            [pltpu.VMEM((k, dim), jnp.float32)] * ring,
            [pltpu.SemaphoreType.DMA] * ring,
            [pltpu.SemaphoreType.DMA] * ring,
        ],
    )
    def gather_kernel(idx_hbm, table_hbm, out_hbm, idx_v, bufs, gsems, ssems):
        wid = lax.axis_index("s") * NC + lax.axis_index("c")
        base = wid * bpw
        pltpu.sync_copy(idx_hbm.at[wid], idx_v)

        def start_g(j, t):
            pltpu.async_copy(table_hbm.at[idx_v.at[j]], bufs[t], gsems[t])

        def wait_g(t):
            pltpu.make_async_copy(
                table_hbm.at[idx_v.at[0]], bufs[t], gsems[t]).wait()

        def start_s(j, t):
            pltpu.async_copy(
                bufs[t], out_hbm.at[pl.ds(base + j * k, k)], ssems[t])

        def wait_s(t):
            pltpu.make_async_copy(
                bufs[t], out_hbm.at[pl.ds(base, k)], ssems[t]).wait()

        for t in range(ring):
            start_g(t, t)

        @pl.loop(0, nchunk // ring)
        def _round(i):
            j = i * ring
            for t in range(ring):
                wait_g(t)
                start_s(j + t, t)
            for t in range(ring):
                @pl.when(j + ring + t < nchunk)
                def _refill():
                    wait_s(t)
                    start_g(j + ring + t, t)

        for t in range(ring):
            wait_s(t)

    return gather_kernel


def kernel(idx, embedding):
    if idx.ndim == 2:
        idx = jnp.squeeze(idx, axis=-1)
    batch = idx.shape[0]
    vocab, dim = embedding.shape
    k, ring = 2, 4
    idx3 = idx.astype(jnp.int32).reshape(NW, (batch // NW) // k, k)
    return _make_gather(batch, vocab, dim, k, ring)(idx3, embedding)
